# Initial kernel scaffold; baseline (speedup 1.0000x reference)
import jax
import jax.numpy as jnp
from jax.experimental import pallas as pl


def kernel(indices, table):
    raise NotImplementedError("write your pallas kernel here")


# speedup vs baseline: 2.9664x; 2.9664x over previous
import functools

import jax
import jax.numpy as jnp
from jax import lax
from jax.experimental import pallas as pl
from jax.experimental.pallas import tpu as pltpu
from jax.experimental.pallas import tpu_sc as plsc

EMB = 128
NC = 2
NS = 16
NW = NC * NS
CHUNK = 128


def _emb_body(n_chunks, table_hbm, idx_hbm, out_hbm, idx_v, rows_v, gsem):
    wid = lax.axis_index("s") * NC + lax.axis_index("c")
    base = wid * n_chunks * CHUNK
    pltpu.sync_copy(idx_hbm.at[wid], idx_v)

    def body(j, carry):
        pltpu.async_copy(table_hbm.at[idx_v.at[j]], rows_v, gsem).wait()
        pltpu.sync_copy(rows_v, out_hbm.at[pl.ds(base + j * CHUNK, CHUNK)])
        return carry

    lax.fori_loop(0, n_chunks, body, 0)


@functools.partial(jax.jit, static_argnames=("n_chunks",))
def _emb_call(table, idx3, n_chunks):
    total = NW * n_chunks * CHUNK
    fn = pl.kernel(
        functools.partial(_emb_body, n_chunks),
        mesh=plsc.VectorSubcoreMesh(core_axis_name="c", subcore_axis_name="s"),
        out_type=jax.ShapeDtypeStruct((total, EMB), jnp.float32),
        scratch_types=[
            pltpu.VMEM((n_chunks, CHUNK), jnp.int32),
            pltpu.VMEM((CHUNK, EMB), jnp.float32),
            pltpu.SemaphoreType.DMA,
        ],
    )
    return fn(table, idx3)


def kernel(indices, table):
    b, s = indices.shape
    total = b * s
    flat = indices.reshape(-1).astype(jnp.int32)
    per_w = total // NW
    n_chunks = per_w // CHUNK
    assert per_w % CHUNK == 0 and total % NW == 0
    idx3 = flat.reshape(NW, n_chunks, CHUNK)
    out = _emb_call(table, idx3, n_chunks)
    return out.reshape(b, s, EMB)


# speedup vs baseline: 3.3422x; 1.1267x over previous
import functools

import jax
import jax.numpy as jnp
from jax import lax
from jax.experimental import pallas as pl
from jax.experimental.pallas import tpu as pltpu
from jax.experimental.pallas import tpu_sc as plsc

EMB = 128
NC = 2
NS = 16
NW = NC * NS
CHUNK = 128
NBUF = 5
LOOK = 3


def _emb_body(n_chunks, table_hbm, idx_hbm, out_hbm, idx_v, rows_v, *sems):
    gsems = sems[:NBUF]
    ssems = sems[NBUF:]
    wid = lax.axis_index("s") * NC + lax.axis_index("c")
    base = wid * n_chunks * CHUNK
    pltpu.sync_copy(idx_hbm.at[wid], idx_v)

    def fire_gather(j, b):
        pltpu.async_copy(table_hbm.at[idx_v.at[j]], rows_v.at[b], gsems[b])

    def wait_gather(j, b):
        pltpu.make_async_copy(
            table_hbm.at[idx_v.at[j]], rows_v.at[b], gsems[b]).wait()

    def out_slice(j):
        return out_hbm.at[pl.ds(base + j * CHUNK, CHUNK)]

    def fire_store(j, b):
        pltpu.async_copy(rows_v.at[b], out_slice(j), ssems[b])

    def wait_store(j, b):
        pltpu.make_async_copy(rows_v.at[b], out_slice(j), ssems[b]).wait()

    for j in range(LOOK):
        fire_gather(j, j % NBUF)
    for j in range(NBUF - LOOK):
        fire_gather(j + LOOK, (j + LOOK) % NBUF)
        wait_gather(j, j % NBUF)
        fire_store(j, j % NBUF)

    j0 = NBUF - LOOK
    steady = n_chunks - j0 - LOOK
    assert steady % NBUF == 0

    def outer(g, carry):
        jg = j0 + g * NBUF
        for r in range(NBUF):
            j = jg + r
            b = (j0 + r) % NBUF
            bf = (j0 + r + LOOK) % NBUF
            wait_store(j - (NBUF - LOOK), bf)
            fire_gather(j + LOOK, bf)
            wait_gather(j, b)
            fire_store(j, b)
        return carry

    lax.fori_loop(0, steady // NBUF, outer, 0)

    for j in range(n_chunks - LOOK, n_chunks):
        b = j % NBUF
        wait_store(j - (NBUF - LOOK), (j + LOOK) % NBUF)
        wait_gather(j, b)
        fire_store(j, b)
    for j in range(n_chunks - (NBUF - LOOK), n_chunks):
        wait_store(j, j % NBUF)


@functools.partial(jax.jit, static_argnames=("n_chunks",))
def _emb_call(table, idx3, n_chunks):
    total = NW * n_chunks * CHUNK
    fn = pl.kernel(
        functools.partial(_emb_body, n_chunks),
        mesh=plsc.VectorSubcoreMesh(core_axis_name="c", subcore_axis_name="s"),
        out_type=jax.ShapeDtypeStruct((total, EMB), jnp.float32),
        scratch_types=[
            pltpu.VMEM((n_chunks, CHUNK), jnp.int32),
            pltpu.VMEM((NBUF, CHUNK, EMB), jnp.float32),
        ] + [pltpu.SemaphoreType.DMA] * (2 * NBUF),
    )
    return fn(table, idx3)


def kernel(indices, table):
    b, s = indices.shape
    total = b * s
    flat = indices.reshape(-1).astype(jnp.int32)
    per_w = total // NW
    n_chunks = per_w // CHUNK
    assert per_w % CHUNK == 0 and total % NW == 0
    idx3 = flat.reshape(NW, n_chunks, CHUNK)
    out = _emb_call(table, idx3, n_chunks)
    return out.reshape(b, s, EMB)
